# baseline (speedup 1.0000x reference)
import functools

import jax
import jax.numpy as jnp
from jax import lax
from jax.experimental import pallas as pl
from jax.experimental.pallas import tpu as pltpu
from jax.experimental.pallas import tpu_sc as plsc

_T, _D, _H, _E = 2048, 1024, 512, 64
_BM = 64
_NW = 32
_CHUNK = _T // _NW


def _route_body(x_ref, wg_ref, slot_ref, off_ref):
    x = x_ref[...]
    wg = wg_ref[...]
    logits = jnp.dot(x, wg, preferred_element_type=jnp.float32)
    lane = lax.broadcasted_iota(jnp.int32, (_T, _E), 1)
    m = jnp.max(logits, axis=1, keepdims=True)
    eid = jnp.min(jnp.where(logits == m, lane, _E), axis=1, keepdims=True)
    oh = (lane == eid).astype(jnp.float32)
    counts = jnp.sum(oh, axis=0, keepdims=True)
    jj = lax.broadcasted_iota(jnp.int32, (_E, 128), 0)
    ll = lax.broadcasted_iota(jnp.int32, (_E, 128), 1)
    lt = (jj < ll).astype(jnp.float32)
    off128 = jnp.dot(counts, lt, preferred_element_type=jnp.float32)
    off_ref[...] = off128.astype(jnp.int32)
    off_e = off128[:, :_E]
    ri = lax.broadcasted_iota(jnp.int32, (128, 128), 0)
    ci = lax.broadcasted_iota(jnp.int32, (128, 128), 1)
    ls = (ci < ri).astype(jnp.float32)
    prefix = jnp.zeros((1, _E), jnp.float32)
    for c in range(_T // 128):
        ohc = oh[c * 128:(c + 1) * 128, :]
        rk = jnp.dot(ls, ohc, preferred_element_type=jnp.float32) + prefix
        slot_rows = jnp.sum(ohc * (rk + off_e), axis=1, keepdims=True)
        slot_ref[c * 128:(c + 1) * 128, :] = slot_rows.astype(jnp.int32)
        prefix = prefix + jnp.sum(ohc, axis=0, keepdims=True)


def _route(x, w_gate):
    return pl.pallas_call(
        _route_body,
        out_shape=[
            jax.ShapeDtypeStruct((_T, 1), jnp.int32),
            jax.ShapeDtypeStruct((1, 128), jnp.int32),
        ],
    )(x, w_gate)


def _gmm_body(off_ref, xs_ref, w1_ref, w2_ref, out_ref):
    e = pl.program_id(0)

    @pl.when(e == 0)
    def _():
        out_ref[...] = jnp.zeros_like(out_ref)

    start = off_ref[e]
    end = off_ref[e + 1]
    s0 = (start // _BM) * _BM
    nch = (end - s0 + _BM - 1) // _BM

    def body(j, carry):
        s = pl.multiple_of(s0 + j * _BM, _BM)
        rows = xs_ref[pl.ds(s, _BM), :]
        ids = s + lax.broadcasted_iota(jnp.int32, (_BM, 1), 0)
        msk = ((ids >= start) & (ids < end)).astype(jnp.float32)
        rows = rows * msk
        h1 = jnp.dot(rows, w1_ref[0], preferred_element_type=jnp.float32)
        o = jnp.dot(h1, w2_ref[0], preferred_element_type=jnp.float32)
        out_ref[pl.ds(s, _BM), :] += o
        return carry

    lax.fori_loop(0, nch, body, 0)


def _gmm(offs, xs, w1, w2):
    grid_spec = pltpu.PrefetchScalarGridSpec(
        num_scalar_prefetch=1,
        grid=(_E,),
        in_specs=[
            pl.BlockSpec((_T, _D), lambda e, off: (0, 0)),
            pl.BlockSpec((1, _D, _H), lambda e, off: (e, 0, 0)),
            pl.BlockSpec((1, _H, _D), lambda e, off: (e, 0, 0)),
        ],
        out_specs=pl.BlockSpec((_T, _D), lambda e, off: (0, 0)),
    )
    return pl.pallas_call(
        _gmm_body,
        grid_spec=grid_spec,
        out_shape=jax.ShapeDtypeStruct((_T, _D), jnp.float32),
        compiler_params=pltpu.CompilerParams(
            dimension_semantics=("arbitrary",)),
    )(offs, xs, w1, w2)


@functools.lru_cache(maxsize=None)
def _sc_kernels():
    mesh = plsc.VectorSubcoreMesh(core_axis_name="c", subcore_axis_name="s")
    deco = functools.partial(
        pl.kernel,
        mesh=mesh,
        out_type=jax.ShapeDtypeStruct((_T, _D), jnp.float32),
        scratch_types=[
            pltpu.VMEM((_CHUNK,), jnp.int32),
            pltpu.VMEM((_CHUNK, _D), jnp.float32),
            pltpu.SemaphoreType.DMA,
        ],
    )

    @deco
    def dispatch(slot_hbm, x_hbm, out_hbm, idx_v, rows_v, sem):
        wid = lax.axis_index("s") * 2 + lax.axis_index("c")
        base = wid * _CHUNK
        pltpu.sync_copy(slot_hbm.at[pl.ds(base, _CHUNK)], idx_v)
        pltpu.sync_copy(x_hbm.at[pl.ds(base, _CHUNK)], rows_v)
        pltpu.async_copy(rows_v, out_hbm.at[idx_v], sem).wait()

    @deco
    def combine(slot_hbm, src_hbm, y_hbm, idx_v, rows_v, sem):
        wid = lax.axis_index("s") * 2 + lax.axis_index("c")
        base = wid * _CHUNK
        pltpu.sync_copy(slot_hbm.at[pl.ds(base, _CHUNK)], idx_v)
        pltpu.async_copy(src_hbm.at[idx_v], rows_v, sem).wait()
        pltpu.sync_copy(rows_v, y_hbm.at[pl.ds(base, _CHUNK)])

    return dispatch, combine


def kernel(x, w_gate, w1, w2):
    slot2d, off2d = _route(x, w_gate)
    slot = slot2d.reshape(_T)
    offs = off2d.reshape(128)
    dispatch, combine = _sc_kernels()
    xs = dispatch(slot, x)
    out_sorted = _gmm(offs, xs, w1, w2)
    return combine(slot, out_sorted)


# speedup vs baseline: 1.1507x; 1.1507x over previous
import functools

import jax
import jax.numpy as jnp
from jax import lax
from jax.experimental import pallas as pl
from jax.experimental.pallas import tpu as pltpu
from jax.experimental.pallas import tpu_sc as plsc

_T, _D, _H, _E = 2048, 1024, 512, 64
_BM = 128
_EPG = 4
_NW = 32
_CHUNK = _T // _NW


def _route_body(x_ref, wg_ref, slot_ref, off_ref):
    x = x_ref[...]
    wg = wg_ref[...]
    logits = jnp.dot(x, wg, preferred_element_type=jnp.float32)
    lane = lax.broadcasted_iota(jnp.int32, (_T, _E), 1)
    m = jnp.max(logits, axis=1, keepdims=True)
    eid = jnp.min(jnp.where(logits == m, lane, _E), axis=1, keepdims=True)
    oh = (lane == eid).astype(jnp.float32)
    counts = jnp.sum(oh, axis=0, keepdims=True)
    jj = lax.broadcasted_iota(jnp.int32, (_E, 128), 0)
    ll = lax.broadcasted_iota(jnp.int32, (_E, 128), 1)
    lt = (jj < ll).astype(jnp.float32)
    off128 = jnp.dot(counts, lt, preferred_element_type=jnp.float32)
    off_ref[...] = off128.astype(jnp.int32)
    off_e = off128[:, :_E]
    ri = lax.broadcasted_iota(jnp.int32, (128, 128), 0)
    ci = lax.broadcasted_iota(jnp.int32, (128, 128), 1)
    ls = (ci < ri).astype(jnp.float32)
    prefix = jnp.zeros((1, _E), jnp.float32)
    for c in range(_T // 128):
        ohc = oh[c * 128:(c + 1) * 128, :]
        rk = jnp.dot(ls, ohc, preferred_element_type=jnp.float32) + prefix
        slot_rows = jnp.sum(ohc * (rk + off_e), axis=1, keepdims=True)
        slot_ref[c * 128:(c + 1) * 128, :] = slot_rows.astype(jnp.int32)
        prefix = prefix + jnp.sum(ohc, axis=0, keepdims=True)


def _route(x, w_gate):
    return pl.pallas_call(
        _route_body,
        out_shape=[
            jax.ShapeDtypeStruct((_T, 1), jnp.int32),
            jax.ShapeDtypeStruct((1, 128), jnp.int32),
        ],
    )(x, w_gate)


def _gmm_body(off_ref, xs_ref, w1_ref, w2_ref, out_ref):
    g = pl.program_id(0)

    @pl.when(g == 0)
    def _():
        out_ref[...] = jnp.zeros_like(out_ref)

    for k in range(_EPG):
        e = g * _EPG + k
        start = off_ref[e]
        end = off_ref[e + 1]
        s0 = (start // _BM) * _BM
        nch = (end - s0 + _BM - 1) // _BM

        def body(j, carry, k=k, start=start, end=end, s0=s0):
            s = pl.multiple_of(s0 + j * _BM, _BM)
            rows = xs_ref[pl.ds(s, _BM), :]
            ids = s + lax.broadcasted_iota(jnp.int32, (_BM, 1), 0)
            msk = ((ids >= start) & (ids < end)).astype(jnp.float32)
            rows = rows * msk
            h1 = jnp.dot(rows, w1_ref[k], preferred_element_type=jnp.float32)
            o = jnp.dot(h1, w2_ref[k], preferred_element_type=jnp.float32)
            out_ref[pl.ds(s, _BM), :] += o
            return carry

        lax.fori_loop(0, nch, body, 0)


def _gmm(offs, xs, w1, w2):
    grid_spec = pltpu.PrefetchScalarGridSpec(
        num_scalar_prefetch=1,
        grid=(_E // _EPG,),
        in_specs=[
            pl.BlockSpec((_T, _D), lambda e, off: (0, 0)),
            pl.BlockSpec((_EPG, _D, _H), lambda e, off: (e, 0, 0)),
            pl.BlockSpec((_EPG, _H, _D), lambda e, off: (e, 0, 0)),
        ],
        out_specs=pl.BlockSpec((_T, _D), lambda e, off: (0, 0)),
    )
    return pl.pallas_call(
        _gmm_body,
        grid_spec=grid_spec,
        out_shape=jax.ShapeDtypeStruct((_T, _D), jnp.float32),
        compiler_params=pltpu.CompilerParams(
            dimension_semantics=("arbitrary",)),
    )(offs, xs, w1, w2)


@functools.lru_cache(maxsize=None)
def _sc_kernels():
    mesh = plsc.VectorSubcoreMesh(core_axis_name="c", subcore_axis_name="s")
    deco = functools.partial(
        pl.kernel,
        mesh=mesh,
        out_type=jax.ShapeDtypeStruct((_T, _D), jnp.float32),
        scratch_types=[
            pltpu.VMEM((_CHUNK,), jnp.int32),
            pltpu.VMEM((_CHUNK, _D), jnp.float32),
            pltpu.SemaphoreType.DMA,
        ],
    )

    @deco
    def dispatch(slot_hbm, x_hbm, out_hbm, idx_v, rows_v, sem):
        wid = lax.axis_index("s") * 2 + lax.axis_index("c")
        base = wid * _CHUNK
        pltpu.sync_copy(slot_hbm.at[pl.ds(base, _CHUNK)], idx_v)
        pltpu.sync_copy(x_hbm.at[pl.ds(base, _CHUNK)], rows_v)
        pltpu.async_copy(rows_v, out_hbm.at[idx_v], sem).wait()

    @deco
    def combine(slot_hbm, src_hbm, y_hbm, idx_v, rows_v, sem):
        wid = lax.axis_index("s") * 2 + lax.axis_index("c")
        base = wid * _CHUNK
        pltpu.sync_copy(slot_hbm.at[pl.ds(base, _CHUNK)], idx_v)
        pltpu.async_copy(src_hbm.at[idx_v], rows_v, sem).wait()
        pltpu.sync_copy(rows_v, y_hbm.at[pl.ds(base, _CHUNK)])

    return dispatch, combine


def kernel(x, w_gate, w1, w2):
    slot2d, off2d = _route(x, w_gate)
    slot = slot2d.reshape(_T)
    offs = off2d.reshape(128)
    dispatch, combine = _sc_kernels()
    xs = dispatch(slot, x)
    out_sorted = _gmm(offs, xs, w1, w2)
    return combine(slot, out_sorted)
